# baseline (speedup 1.0000x reference)
import functools

import jax
import jax.numpy as jnp
from jax import lax
from jax.experimental import pallas as pl
from jax.experimental.pallas import tpu as pltpu
from jax.experimental.pallas import tpu_sc as plsc

D = 1024
L = 16
NC, NS = 2, 16
NW = NC * NS
CHUNK = 16
BLK = 1024


def _gather_body(tok_idx_hbm, tok_tab, out_hbm,
                 tok_idx_v, t0_v, t1_v, sem_g0, sem_g1, sem_o0, sem_o1):
  per_w = tok_idx_hbm.shape[0] // NW
  n_chunks = per_w // CHUNK
  wid = lax.axis_index("s") * NC + lax.axis_index("c")
  base = wid * per_w

  pltpu.sync_copy(tok_idx_hbm.at[pl.ds(base, per_w)], tok_idx_v)

  bufs = ((t0_v, sem_g0, sem_o0), (t1_v, sem_g1, sem_o1))

  def issue_gather(rb, t_v, sem):
    pltpu.async_copy(tok_tab.at[tok_idx_v.at[pl.ds(rb, CHUNK)]], t_v, sem)

  def drain_gather(rb, t_v, sem):
    pltpu.make_async_copy(tok_tab.at[tok_idx_v.at[pl.ds(rb, CHUNK)]], t_v,
                          sem).wait()

  issue_gather(0, t0_v, sem_g0)

  def pair_body(i, _):
    for b in (0, 1):
      t_v, sem_g, sem_o = bufs[b]
      tn_v, sem_gn, sem_on = bufs[1 - b]
      c = i * 2 + b
      rb = c * CHUNK

      drain_gather(rb, t_v, sem_g)

      @pl.when(c >= 1)
      def _():
        pltpu.make_async_copy(tn_v, out_hbm.at[pl.ds(base + rb - CHUNK,
                                                     CHUNK)], sem_on).wait()

      @pl.when(c + 1 < n_chunks)
      def _():
        issue_gather(rb + CHUNK, tn_v, sem_gn)

      pltpu.async_copy(t_v, out_hbm.at[pl.ds(base + rb, CHUNK)], sem_o)
    return ()

  lax.fori_loop(0, n_chunks // 2, pair_body, (), unroll=False)

  pltpu.make_async_copy(t1_v, out_hbm.at[pl.ds(base + per_w - CHUNK, CHUNK)],
                        sem_o1).wait()


def _ln_math(g_ref, pos_ref, segf_ref, seg_tab_ref, gamma_ref, beta_ref,
             out_ref):
  st0 = seg_tab_ref[0, :]
  st1 = seg_tab_ref[1, :]
  segf = segf_ref[...]
  x = g_ref[...] + pos_ref[...] + (st0[None, :] + segf * (st1 - st0)[None, :])
  mean = jnp.mean(x, axis=-1, keepdims=True)
  xc = x - mean
  var = jnp.mean(xc * xc, axis=-1, keepdims=True)
  inv = lax.rsqrt(var + 1e-5)
  out_ref[...] = xc * inv * gamma_ref[...] + beta_ref[...]


def _ln_body(g_ref, pos_ref, segf_ref, seg_tab_ref, gamma_ref, beta_ref,
             out_ref):
  _ln_math(g_ref, pos_ref, segf_ref, seg_tab_ref, gamma_ref, beta_ref,
           out_ref)


def _ln_body_acc(g_ref, pos_ref, segf_ref, seg_tab_ref, gamma_ref, beta_ref,
                 prev_ref, out_ref):
  del prev_ref
  _ln_math(g_ref, pos_ref, segf_ref, seg_tab_ref, gamma_ref, beta_ref,
           out_ref)


@jax.jit
def kernel(input_ids, segment_ids, token_table, position_table, segment_table,
           ln_gamma, ln_beta):
  b, s = input_ids.shape
  t = b * s
  tok_idx = input_ids.reshape(t).astype(jnp.int32)
  segf = segment_ids.reshape(t, 1).astype(jnp.float32)

  mesh = plsc.VectorSubcoreMesh(core_axis_name="c", subcore_axis_name="s",
                                num_cores=NC, num_subcores=NS)
  half = t // 2
  per_w = half // NW
  gather_rows = functools.partial(
      pl.kernel,
      out_type=jax.ShapeDtypeStruct((half, D), jnp.float32),
      mesh=mesh,
      compiler_params=pltpu.CompilerParams(needs_layout_passes=False),
      scratch_types=[
          pltpu.VMEM((per_w,), jnp.int32),
          pltpu.VMEM((CHUNK, D), jnp.float32),
          pltpu.VMEM((CHUNK, D), jnp.float32),
          pltpu.SemaphoreType.DMA,
          pltpu.SemaphoreType.DMA,
          pltpu.SemaphoreType.DMA,
          pltpu.SemaphoreType.DMA,
      ],
  )(_gather_body)
  g0 = gather_rows(tok_idx[:half], token_table)
  g1 = gather_rows(tok_idx[half:], token_table)

  hb = b // 2
  sb = s // BLK
  gamma2 = ln_gamma.reshape(1, D)
  beta2 = ln_beta.reshape(1, D)

  common_in_specs = [
      pl.BlockSpec((BLK, D), lambda i, j: (j * sb + i, 0)),
      pl.BlockSpec((BLK, D), lambda i, j: (i, 0)),
      pl.BlockSpec((BLK, 1), lambda i, j: (j * sb + i, 0)),
      pl.BlockSpec((2, D), lambda i, j: (0, 0)),
      pl.BlockSpec((1, D), lambda i, j: (0, 0)),
      pl.BlockSpec((1, D), lambda i, j: (0, 0)),
  ]
  o0 = pl.pallas_call(
      _ln_body,
      grid=(sb, hb),
      in_specs=common_in_specs,
      out_specs=pl.BlockSpec((BLK, D), lambda i, j: (j * sb + i, 0)),
      out_shape=jax.ShapeDtypeStruct((t, D), jnp.float32),
  )(g0, position_table, segf[:half], segment_table, gamma2, beta2)
  hsb = hb * sb
  out = pl.pallas_call(
      _ln_body_acc,
      grid=(sb, hb),
      in_specs=common_in_specs + [pl.BlockSpec((BLK, D), lambda i, j: (0, 0))],
      out_specs=pl.BlockSpec((BLK, D), lambda i, j: (hsb + j * sb + i, 0)),
      out_shape=jax.ShapeDtypeStruct((t, D), jnp.float32),
      input_output_aliases={6: 0},
  )(g1, position_table, segf[half:], segment_table, gamma2, beta2, o0)
  return out.reshape(b, s, D)
